# baseline (speedup 1.0000x reference)
import functools

import jax
import jax.numpy as jnp
from jax import lax
from jax.experimental import pallas as pl
from jax.experimental.pallas import tpu as pltpu
from jax.experimental.pallas import tpu_sc as plsc

KNN = 20
NEG_SLOPE = 0.2

SC_CORES = 2
SC_SUBCORES = 16
SC_WORKERS = SC_CORES * SC_SUBCORES
SC_ROWS_PER_CHUNK = 4


def _lrelu(v):
    return jnp.maximum(v, NEG_SLOPE * v)


def _feat_body(xt_ref, w1_ref, w2a_ref, w2b_ref, yt_ref, zt_ref, wt_ref):
    xt = xt_ref[0]
    dn = (((1,), (1,)), ((), ()))
    yt = lax.dot_general(xt, w1_ref[...], dn, preferred_element_type=jnp.float32)
    yt_ref[0] = yt
    yt16 = yt.astype(jnp.bfloat16)
    zt_ref[0] = lax.dot_general(yt16, w2a_ref[...], dn,
                                preferred_element_type=jnp.float32)
    wt_ref[0] = lax.dot_general(yt16, w2b_ref[...], dn,
                                preferred_element_type=jnp.float32)


def _knn_body(ytf_ref, ytb_ref, idx_ref):
    yf = ytf_ref[0]
    yb = ytb_ref[0]
    m, c = yb.shape
    n = yf.shape[0]
    dn = (((1,), (1,)), ((), ()))
    inner = lax.dot_general(yb.astype(jnp.bfloat16), yf.astype(jnp.bfloat16),
                            dn, preferred_element_type=jnp.float32)
    sq_row = jnp.sum(yf * yf, axis=1)[None, :]
    sq_col = jnp.sum(yb * yb, axis=1)[:, None]
    d = (2.0 * inner - sq_col) - sq_row
    col = lax.broadcasted_iota(jnp.int32, (m, n), 1)
    t = jnp.full((m, 1), jnp.inf, jnp.float32)
    a = jnp.full((m, 1), -1, jnp.int32)
    for k in range(KNN):
        rem = jnp.where((d < t) | ((d == t) & (col > a)), d, -jnp.inf)
        t = jnp.max(rem, axis=1, keepdims=True)
        a = jnp.min(jnp.where(rem == t, col, n), axis=1, keepdims=True)
        idx_ref[0, :, k:k + 1] = a


def _sc_attention(zt, yt, wt, gidx, bn, c):
    rows_per_worker = bn // SC_WORKERS
    r = SC_ROWS_PER_CHUNK
    n_chunks = rows_per_worker // r
    mesh = plsc.VectorSubcoreMesh(core_axis_name="c", subcore_axis_name="s")

    @functools.partial(
        pl.kernel,
        mesh=mesh,
        out_type=jax.ShapeDtypeStruct((bn, c), jnp.float32),
        scratch_types=[
            pltpu.VMEM((r * KNN,), jnp.int32),
            pltpu.VMEM((r * KNN, c), jnp.float32),
            pltpu.VMEM((r * KNN, c), jnp.float32),
            pltpu.VMEM((r, c), jnp.float32),
            pltpu.VMEM((r, c), jnp.float32),
            pltpu.SemaphoreType.DMA,
            pltpu.SemaphoreType.DMA,
        ],
    )
    def att(zt_hbm, yt_hbm, wt_hbm, gidx_hbm, out_hbm,
            idx_v, z_v, y_v, w_v, o_v, sem_z, sem_y):
        wid = lax.axis_index("s") * SC_CORES + lax.axis_index("c")
        base0 = wid * rows_per_worker

        def chunk(ci, carry):
            base = base0 + ci * r
            pltpu.sync_copy(gidx_hbm.at[pl.ds(base * KNN, r * KNN)], idx_v)
            cp_z = pltpu.async_copy(zt_hbm.at[idx_v], z_v, sem_z)
            cp_y = pltpu.async_copy(yt_hbm.at[idx_v], y_v, sem_y)
            pltpu.sync_copy(wt_hbm.at[pl.ds(base, r)], w_v)
            cp_z.wait()
            cp_y.wait()

            def row(ri, carry2):
                def chan(c16, carry3):
                    co = c16 * 16
                    wv = w_v[ri, pl.ds(co, 16)]
                    es = []
                    mx = jnp.full((16,), -jnp.inf, jnp.float32)
                    for k in range(KNN):
                        e = _lrelu(z_v[ri * KNN + k, pl.ds(co, 16)] + wv)
                        es.append(e)
                        mx = jnp.maximum(mx, e)
                    s = jnp.zeros((16,), jnp.float32)
                    o = jnp.zeros((16,), jnp.float32)
                    for k in range(KNN):
                        p = jnp.exp(es[k] - mx)
                        s = s + p
                        o = o + p * y_v[ri * KNN + k, pl.ds(co, 16)]
                    o_v[ri, pl.ds(co, 16)] = _lrelu(o / s)
                    return carry3

                return lax.fori_loop(0, c // 16, chan, carry2)

            lax.fori_loop(0, r, row, 0)
            pltpu.sync_copy(o_v, out_hbm.at[pl.ds(base, r)])
            return carry

        lax.fori_loop(0, n_chunks, chunk, 0)

    return att(zt, yt, wt, gidx)


def kernel(x, W1, W2):
    b, cin, n = x.shape
    c = W1.shape[0]
    xt = jnp.transpose(x, (0, 2, 1)).astype(jnp.bfloat16)
    w1_16 = W1.astype(jnp.bfloat16)
    w2a = W2[:, :c].astype(jnp.bfloat16)
    w2b = W2[:, c:].astype(jnp.bfloat16)

    yt, zt, wt = pl.pallas_call(
        _feat_body,
        grid=(b,),
        in_specs=[
            pl.BlockSpec((1, n, cin), lambda i: (i, 0, 0)),
            pl.BlockSpec((c, cin), lambda i: (0, 0)),
            pl.BlockSpec((c, c), lambda i: (0, 0)),
            pl.BlockSpec((c, c), lambda i: (0, 0)),
        ],
        out_specs=[
            pl.BlockSpec((1, n, c), lambda i: (i, 0, 0)),
            pl.BlockSpec((1, n, c), lambda i: (i, 0, 0)),
            pl.BlockSpec((1, n, c), lambda i: (i, 0, 0)),
        ],
        out_shape=[
            jax.ShapeDtypeStruct((b, n, c), jnp.float32),
            jax.ShapeDtypeStruct((b, n, c), jnp.float32),
            jax.ShapeDtypeStruct((b, n, c), jnp.float32),
        ],
    )(xt, w1_16, w2a, w2b)

    m = 256
    outs = []
    for i in range(b):
        yt_i = lax.slice_in_dim(yt, i, i + 1, axis=0)
        gidx_i = pl.pallas_call(
            _knn_body,
            grid=(1, n // m),
            in_specs=[
                pl.BlockSpec((1, n, c), lambda i, j: (i, 0, 0)),
                pl.BlockSpec((1, m, c), lambda i, j: (i, j, 0)),
            ],
            out_specs=pl.BlockSpec((1, m, KNN), lambda i, j: (i, j, 0)),
            out_shape=jax.ShapeDtypeStruct((1, n, KNN), jnp.int32),
        )(yt_i, yt_i)
        outs.append(_sc_attention(
            zt[i], yt[i], wt[i], gidx_i.reshape(n * KNN), n, c))
    outt = jnp.stack(outs)
    return jnp.transpose(outt, (0, 2, 1))


# speedup vs baseline: 1.4791x; 1.4791x over previous
import functools

import jax
import jax.numpy as jnp
from jax import lax
from jax.experimental import pallas as pl
from jax.experimental.pallas import tpu as pltpu
from jax.experimental.pallas import tpu_sc as plsc

KNN = 20
NEG_SLOPE = 0.2

SC_CORES = 2
SC_SUBCORES = 16
SC_WORKERS = SC_CORES * SC_SUBCORES
SC_ROWS_PER_CHUNK = 4


def _lrelu(v):
    return jnp.maximum(v, NEG_SLOPE * v)


def _feat_body(xt_ref, w1_ref, w2a_ref, w2b_ref, yt_ref, zt_ref, wt_ref):
    xt = xt_ref[0]
    dn = (((1,), (1,)), ((), ()))
    yt = lax.dot_general(xt, w1_ref[...], dn, preferred_element_type=jnp.float32)
    yt_ref[0] = yt
    yt16 = yt.astype(jnp.bfloat16)
    zt_ref[0] = lax.dot_general(yt16, w2a_ref[...], dn,
                                preferred_element_type=jnp.float32)
    wt_ref[0] = lax.dot_general(yt16, w2b_ref[...], dn,
                                preferred_element_type=jnp.float32)


def _knn_body(ytf_ref, ytb_ref, idx_ref, d_ref):
    b = pl.program_id(0)
    yf = ytf_ref[0]
    yb = ytb_ref[0]
    m, c = yb.shape
    n = yf.shape[0]
    dn = (((1,), (1,)), ((), ()))
    inner = lax.dot_general(yb.astype(jnp.bfloat16), yf.astype(jnp.bfloat16),
                            dn, preferred_element_type=jnp.float32)
    sq_row = jnp.sum(yf * yf, axis=1)[None, :]
    sq_col = jnp.sum(yb * yb, axis=1)[:, None]
    d_ref[...] = (2.0 * inner - sq_col) - sq_row
    col = lax.broadcasted_iota(jnp.int32, (m, n), 1)
    for k in range(KNN):
        d = d_ref[...]
        mx = jnp.max(d, axis=1, keepdims=True)
        cand = jnp.where(d >= mx, col, n)
        a = jnp.min(cand, axis=1, keepdims=True)
        idx_ref[0, :, k:k + 1] = a + b * n
        d_ref[...] = jnp.where(cand == a, -jnp.inf, d)


def _sc_attention(zt, yt, wt, gidx, bn, c):
    rows_per_worker = bn // SC_WORKERS
    r = SC_ROWS_PER_CHUNK
    n_chunks = rows_per_worker // r
    mesh = plsc.VectorSubcoreMesh(core_axis_name="c", subcore_axis_name="s")

    @functools.partial(
        pl.kernel,
        mesh=mesh,
        out_type=jax.ShapeDtypeStruct((bn, c), jnp.float32),
        scratch_types=[
            pltpu.VMEM((r * KNN,), jnp.int32),
            pltpu.VMEM((r * KNN, c), jnp.float32),
            pltpu.VMEM((r * KNN, c), jnp.float32),
            pltpu.VMEM((r, c), jnp.float32),
            pltpu.VMEM((r, c), jnp.float32),
            pltpu.SemaphoreType.DMA,
            pltpu.SemaphoreType.DMA,
        ],
    )
    def att(zt_hbm, yt_hbm, wt_hbm, gidx_hbm, out_hbm,
            idx_v, z_v, y_v, w_v, o_v, sem_z, sem_y):
        wid = lax.axis_index("s") * SC_CORES + lax.axis_index("c")
        base0 = wid * rows_per_worker

        def chunk(ci, carry):
            base = base0 + ci * r
            pltpu.sync_copy(gidx_hbm.at[pl.ds(base * KNN, r * KNN)], idx_v)
            cp_z = pltpu.async_copy(zt_hbm.at[idx_v], z_v, sem_z)
            cp_y = pltpu.async_copy(yt_hbm.at[idx_v], y_v, sem_y)
            pltpu.sync_copy(wt_hbm.at[pl.ds(base, r)], w_v)
            cp_z.wait()
            cp_y.wait()

            def row(ri, carry2):
                def chan(c16, carry3):
                    co = c16 * 16
                    wv = w_v[ri, pl.ds(co, 16)]
                    es = []
                    mx = jnp.full((16,), -jnp.inf, jnp.float32)
                    for k in range(KNN):
                        e = _lrelu(z_v[ri * KNN + k, pl.ds(co, 16)] + wv)
                        es.append(e)
                        mx = jnp.maximum(mx, e)
                    s = jnp.zeros((16,), jnp.float32)
                    o = jnp.zeros((16,), jnp.float32)
                    for k in range(KNN):
                        p = jnp.exp(es[k] - mx)
                        s = s + p
                        o = o + p * y_v[ri * KNN + k, pl.ds(co, 16)]
                    o_v[ri, pl.ds(co, 16)] = _lrelu(o / s)
                    return carry3

                return lax.fori_loop(0, c // 16, chan, carry2)

            lax.fori_loop(0, r, row, 0)
            pltpu.sync_copy(o_v, out_hbm.at[pl.ds(base, r)])
            return carry

        lax.fori_loop(0, n_chunks, chunk, 0)

    return att(zt, yt, wt, gidx)


def kernel(x, W1, W2):
    b, cin, n = x.shape
    c = W1.shape[0]
    xt = jnp.transpose(x, (0, 2, 1)).astype(jnp.bfloat16)
    w1_16 = W1.astype(jnp.bfloat16)
    w2a = W2[:, :c].astype(jnp.bfloat16)
    w2b = W2[:, c:].astype(jnp.bfloat16)

    yt, zt, wt = pl.pallas_call(
        _feat_body,
        grid=(b,),
        in_specs=[
            pl.BlockSpec((1, n, cin), lambda i: (i, 0, 0)),
            pl.BlockSpec((c, cin), lambda i: (0, 0)),
            pl.BlockSpec((c, c), lambda i: (0, 0)),
            pl.BlockSpec((c, c), lambda i: (0, 0)),
        ],
        out_specs=[
            pl.BlockSpec((1, n, c), lambda i: (i, 0, 0)),
            pl.BlockSpec((1, n, c), lambda i: (i, 0, 0)),
            pl.BlockSpec((1, n, c), lambda i: (i, 0, 0)),
        ],
        out_shape=[
            jax.ShapeDtypeStruct((b, n, c), jnp.float32),
            jax.ShapeDtypeStruct((b, n, c), jnp.float32),
            jax.ShapeDtypeStruct((b, n, c), jnp.float32),
        ],
    )(xt, w1_16, w2a, w2b)

    m = 256
    outs = []
    for i in range(b):
        yt_i = lax.slice_in_dim(yt, i, i + 1, axis=0)
        gidx_i = pl.pallas_call(
            _knn_body,
            grid=(1, n // m),
            in_specs=[
                pl.BlockSpec((1, n, c), lambda i, j: (i, 0, 0)),
                pl.BlockSpec((1, m, c), lambda i, j: (i, j, 0)),
            ],
            out_specs=pl.BlockSpec((1, m, KNN), lambda i, j: (i, j, 0)),
            out_shape=jax.ShapeDtypeStruct((1, n, KNN), jnp.int32),
            scratch_shapes=[pltpu.VMEM((m, n), jnp.float32)],
            compiler_params=pltpu.CompilerParams(
                dimension_semantics=("parallel", "parallel")),
        )(yt_i, yt_i)
        outs.append(_sc_attention(
            zt[i], yt[i], wt[i], gidx_i.reshape(n * KNN), n, c))
    outt = jnp.stack(outs)
    return jnp.transpose(outt, (0, 2, 1))
